# baseline (speedup 1.0000x reference)
import numpy as np
import jax
import jax.numpy as jnp
from jax import lax
from jax.experimental import pallas as pl
from jax.experimental.pallas import tpu as pltpu
from jax.experimental.pallas import tpu_sc as plsc

NUM_LEAVES = 4096
NUM_SITES = 256
NS = 4
_IU = np.triu_indices(NS, 1)

LEVEL_PARENTS = [NUM_LEAVES >> (l + 1) for l in range(12)]
LEVEL_OFF = [0]
for _n in LEVEL_PARENTS:
    LEVEL_OFF.append(LEVEL_OFF[-1] + _n)

NQ = 4
NG = 8
SPW = NUM_SITES // NG
LPQ = NUM_LEAVES // NQ
CH = 8
CL = LPQ // CH

CH_COL = [0, 64, 96]
QB_N = [64, 32, 16, 8, 4, 2, 1]
QB_RC = [(0, 0), (0, 64), (0, 96), (0, 112), (32, 0), (32, 16), (32, 32)]


def _cv(x):
    return jnp.full((16,), x, jnp.int32)


def _pack_indices():
    flat = np.arange((2 * NUM_LEAVES - 1) * 16, dtype=np.int64).reshape(-1, 16)
    pi0 = flat.size
    zero = flat.size + NS
    pl_parts, pr_parts = [], []
    for l in range(12):
        n = LEVEL_PARENTS[l]
        base = 0 if l == 0 else NUM_LEAVES + LEVEL_OFF[l - 1]
        seg = flat[base:base + 2 * n].reshape(n, 2, 16)
        pl_parts.append(seg[:, 0, :])
        pr_parts.append(seg[:, 1, :])
    plf = np.concatenate(pl_parts, axis=0).T
    prf = np.concatenate(pr_parts, axis=0).T
    cblocks = []
    for l in range(3):
        n = 64 >> l
        off = LEVEL_OFF[l]
        seg_l = plf[:, off:off + (2048 >> l)].reshape(16, 4, CH, n)
        seg_r = prf[:, off:off + (2048 >> l)].reshape(16, 4, CH, n)
        blk = np.concatenate([seg_l, seg_r], axis=0)
        cblocks.append(blk.transpose(1, 2, 0, 3))
    pqc = np.concatenate(cblocks, axis=3)
    pqc = np.concatenate(
        [pqc, np.full((NQ, CH, 32, 16), zero, np.int64)], axis=3)
    pqb = np.full((NQ, 64, 128), zero, np.int64)
    for i, n in enumerate(QB_N):
        off = LEVEL_OFF[3 + i]
        seg_l = plf[:, off:off + 4 * n].reshape(16, 4, n).transpose(1, 0, 2)
        seg_r = prf[:, off:off + 4 * n].reshape(16, 4, n).transpose(1, 0, 2)
        blk = np.concatenate([seg_l, seg_r], axis=1)
        r0, c0 = QB_RC[i]
        pqb[:, r0:r0 + 32, c0:c0 + n] = blk
    def stage_block(off, n):
        b = np.concatenate([plf[:, off:off + n], prf[:, off:off + n]], axis=0)
        return np.concatenate(
            [b, np.full((32, 16 - n), zero, np.int64)], axis=1)
    pcs = np.full((32, 128), zero, np.int64)
    pcs[:, 0:16] = stage_block(LEVEL_OFF[10], 2)
    pcs[:, 16:32] = stage_block(LEVEL_OFF[11], 1)
    pcs[0:NS, 32] = pi0 + np.arange(NS)
    return (np.concatenate([pqc.reshape(-1), pqb.reshape(-1),
                            pcs.reshape(-1)]).astype(np.int32))


_PACK_IDX = _pack_indices()
_N_PQC = NQ * CH * 32 * 128
_N_PQB = NQ * 64 * 128


def _prep(rates, pi_logits, lengths, operations):
    pi = jax.nn.softmax(pi_logits)
    R = jnp.zeros((NS, NS), rates.dtype).at[_IU].set(rates)
    R = R + R.T
    Q = R * pi[None, :]
    Q = Q - jnp.diag(jnp.sum(Q, axis=1))
    mu = -jnp.sum(pi * jnp.diag(Q))
    Q = Q / mu
    dsq = jnp.sqrt(pi)
    S = (Q * dsq[:, None]) / dsq[None, :]
    S = 0.5 * (S + S.T)
    evals, evecs = jnp.linalg.eigh(S)
    left = evecs / dsq[:, None]
    right = evecs.T * dsq[None, :]
    P = jax.vmap(lambda t: left @ (jnp.exp(evals * t)[:, None] * right))(
        lax.stop_gradient(lengths))
    P = jnp.maximum(P, 0.0).astype(jnp.float32)

    flatP = jnp.concatenate([P.reshape(-1), pi,
                             jnp.zeros((1,), jnp.float32)])
    packed = jnp.take(flatP, jnp.asarray(_PACK_IDX), axis=0)
    pqc = packed[:_N_PQC].reshape(NQ, CH, 32, 128)
    pqb = packed[_N_PQC:_N_PQC + _N_PQB].reshape(NQ, 64, 128)
    pcs = packed[_N_PQC + _N_PQB:].reshape(32, 128)
    return pqc, pqb, pcs


def _do_level(pbuf, rbase, pcol0, npar, load_val, load_sc,
              store_val, store_sc):
    for gi in range(max(1, npar // 16)):
        p0 = gi * 16
        plv = [pbuf[rbase + r, pl.ds(pcol0 + p0, 16)] for r in range(16)]
        prv = [pbuf[rbase + 16 + r, pl.ds(pcol0 + p0, 16)]
               for r in range(16)]

        def body(t, p0=p0, plv=plv, prv=prv):
            ce = (p0 + lax.iota(jnp.int32, 16)) * 2
            lv = [load_val(t, j, ce) for j in range(NS)]
            rv = [load_val(t, j, ce + 1) for j in range(NS)]
            outs = []
            for s in range(NS):
                fl = plv[s * 4] * lv[0]
                fr = prv[s * 4] * rv[0]
                for j in range(1, NS):
                    fl = fl + plv[s * 4 + j] * lv[j]
                    fr = fr + prv[s * 4 + j] * rv[j]
                outs.append(fl * fr)
            m = jnp.maximum(jnp.maximum(outs[0], outs[1]),
                            jnp.maximum(outs[2], outs[3]))
            bits = plsc.bitcast(m, jnp.int32)
            eb = (bits >> 23) & 255
            ok = m > 0.0
            e = jnp.where(ok, eb - 127, 0)
            inv = jnp.where(ok, plsc.bitcast((254 - eb) << 23, jnp.float32),
                            jnp.float32(1.0))
            for s in range(NS):
                store_val(t, s, p0, outs[s] * inv)
            sc = e if load_sc is None else \
                load_sc(t, ce) + load_sc(t, ce + 1) + e
            store_sc(t, p0, sc)

        plsc.parallel_loop(0, SPW, unroll=1)(body)


def _vload(ref):
    return lambda t, j, cvec: plsc.load_gather(ref, [_cv(t + 32 * j), cvec])


def _sload(ref):
    return lambda t, cvec: plsc.load_gather(ref, [_cv(t), cvec])


def _vstore(ref, col0):
    def st(t, s, p0, x):
        ref[t + 32 * s, pl.ds(col0 + p0, 16)] = x
    return st


def _sstore(ref, col0):
    def st(t, p0, x):
        ref[t, pl.ds(col0 + p0, 16)] = x
    return st


def _root_vstore(ref):
    def st(t, s, p0, x):
        ref[t, pl.ds(16 * s, 16)] = x
    return st


def _root_sstore(ref):
    def st(t, p0, x):
        ref[t, pl.ds(64, 16)] = plsc.bitcast(x, jnp.float32)
    return st


def _body(leaf_hbm, pqc_hbm, pqb_hbm, pc_hbm, outv_hbm, outs_hbm,
          lbufa, lbufb, pbufc, pbufb, pcbuf, va, vb, acc, rootv,
          sa, sb, sacc, ov, osb, shv, sema, semb):
    c = lax.axis_index("c")
    s_id = lax.axis_index("s")
    q = s_id // 4
    g = c * 4 + (s_id % 4)
    site0 = g * SPW
    leaf0 = q * LPQ

    pltpu.sync_copy(pqb_hbm.at[q], pbufb)
    pltpu.sync_copy(pc_hbm, pcbuf)

    def leaf_slice(cc):
        return leaf_hbm.at[pl.ds(leaf0 + cc * CL, CL), g]

    def compute_chunk(cc, lbuf):
        def leaf_load(t, j, cvec):
            return plsc.load_gather(lbuf, [cvec, _cv(4 * t + j)])

        _do_level(pbufc, 0, CH_COL[0], 64,
                  leaf_load, None, _vstore(va, 0), _sstore(sa, 0))
        _do_level(pbufc, 0, CH_COL[1], 32,
                  _vload(va), _sload(sa), _vstore(vb, 0), _sstore(sb, 0))
        _do_level(pbufc, 0, CH_COL[2], 16,
                  _vload(vb), _sload(sb),
                  _vstore(acc, cc * 16), _sstore(sacc, cc * 16))

    pltpu.async_copy(leaf_slice(0), lbufa, sema)

    def chunk_pair(p, _):
        cc0 = 2 * p
        pltpu.async_copy(leaf_slice(cc0 + 1), lbufb, semb)
        pltpu.sync_copy(pqc_hbm.at[q, cc0], pbufc)
        pltpu.make_async_copy(leaf_slice(cc0), lbufa, sema).wait()
        compute_chunk(cc0, lbufa)

        @pl.when(p < CH // 2 - 1)
        def _prefetch():
            pltpu.async_copy(leaf_slice(cc0 + 2), lbufa, sema)

        pltpu.sync_copy(pqc_hbm.at[q, cc0 + 1], pbufc)
        pltpu.make_async_copy(leaf_slice(cc0 + 1), lbufb, semb).wait()
        compute_chunk(cc0 + 1, lbufb)
        return 0

    lax.fori_loop(0, CH // 2, chunk_pair, 0)

    chain = [(acc, sacc), (va, sa), (vb, sb), (va, sa),
             (vb, sb), (va, sa), (vb, sb)]
    for i, npar in enumerate(QB_N):
        src_v, src_s = chain[i]
        r0, c0 = QB_RC[i]
        if i < len(QB_N) - 1:
            dst_v, dst_s = chain[i + 1]
            _do_level(pbufb, r0, c0, npar, _vload(src_v), _sload(src_s),
                      _vstore(dst_v, 0), _sstore(dst_s, 0))
        else:
            _do_level(pbufb, r0, c0, npar, _vload(src_v), _sload(src_s),
                      _root_vstore(rootv), _root_sstore(rootv))

    pltpu.sync_copy(rootv, shv.at[s_id])
    plsc.subcore_barrier()

    @pl.when(s_id < 4)
    def _stage_c():
        for k in range(4):
            pltpu.sync_copy(shv.at[s_id + 4 * k], acc.at[pl.ds(32 * k, 32)])

        def spl(row, col):
            return plsc.load_gather(pcbuf, [_cv(row), _cv(col)])

        for h in range(2):
            tvec = lax.iota(jnp.int32, 16) + 16 * h
            vq = [[plsc.load_gather(acc, [tvec + 32 * k, _cv(16 * j)])
                   for j in range(NS)] for k in range(4)]
            scq = [plsc.bitcast(
                plsc.load_gather(acc, [tvec + 32 * k, _cv(64)]), jnp.int32)
                for k in range(4)]
            o1, s1 = [], []
            for i in range(2):
                row = []
                for s in range(NS):
                    fl = spl(s * 4, i) * vq[2 * i][0]
                    fr = spl(16 + s * 4, i) * vq[2 * i + 1][0]
                    for j in range(1, NS):
                        fl = fl + spl(s * 4 + j, i) * vq[2 * i][j]
                        fr = fr + spl(16 + s * 4 + j, i) * vq[2 * i + 1][j]
                    row.append(fl * fr)
                o1.append(row)
                s1.append(scq[2 * i] + scq[2 * i + 1])
            dot = jnp.zeros((16,), jnp.float32)
            for s in range(NS):
                fl = spl(s * 4, 16) * o1[0][0]
                fr = spl(16 + s * 4, 16) * o1[1][0]
                for j in range(1, NS):
                    fl = fl + spl(s * 4 + j, 16) * o1[0][j]
                    fr = fr + spl(16 + s * 4 + j, 16) * o1[1][j]
                dot = dot + spl(s, 32) * (fl * fr)
            ov[pl.ds(16 * h, 16)] = dot
            osb[pl.ds(16 * h, 16)] = (s1[0] + s1[1]).astype(jnp.float32)
        pltpu.sync_copy(ov, outv_hbm.at[pl.ds(site0, SPW)])
        pltpu.sync_copy(osb, outs_hbm.at[pl.ds(site0, SPW)])


def _scratch_types():
    f32, i32 = jnp.float32, jnp.int32
    return [
        pltpu.VMEM((CL, 128), f32),
        pltpu.VMEM((CL, 128), f32),
        pltpu.VMEM((32, 128), f32),
        pltpu.VMEM((64, 128), f32),
        pltpu.VMEM((32, 128), f32),
        pltpu.VMEM((128, 64), f32),
        pltpu.VMEM((128, 32), f32),
        pltpu.VMEM((128, 128), f32),
        pltpu.VMEM((32, 128), f32),
        pltpu.VMEM((32, 64), i32),
        pltpu.VMEM((32, 32), i32),
        pltpu.VMEM((32, 128), i32),
        pltpu.VMEM((SPW,), f32),
        pltpu.VMEM((SPW,), f32),
        pltpu.VMEM_SHARED((16, 32, 128), f32),
        pltpu.SemaphoreType.DMA,
        pltpu.SemaphoreType.DMA,
    ]


def kernel(leaf_data, rates, pi_logits, aligned_branch_lengths, operations):
    pqc, pqb, pcs = _prep(rates, pi_logits, aligned_branch_lengths,
                          operations)
    fn = pl.kernel(
        _body,
        out_type=[jax.ShapeDtypeStruct((NUM_SITES,), jnp.float32),
                  jax.ShapeDtypeStruct((NUM_SITES,), jnp.float32)],
        mesh=plsc.VectorSubcoreMesh(core_axis_name="c", subcore_axis_name="s"),
        scratch_types=_scratch_types(),
        compiler_params=pltpu.CompilerParams(needs_layout_passes=False),
    )
    leaf3 = leaf_data.reshape(NUM_LEAVES, NG, SPW * NS)
    val, sc = fn(leaf3, pqc, pqb, pcs)
    return jnp.mean(jnp.log(val) + sc * jnp.float32(np.log(2.0)))


# speedup vs baseline: 1.5379x; 1.5379x over previous
import numpy as np
import jax
import jax.numpy as jnp
from jax import lax
from jax.experimental import pallas as pl
from jax.experimental.pallas import tpu as pltpu
from jax.experimental.pallas import tpu_sc as plsc

NUM_LEAVES = 4096
NUM_SITES = 256
NS = 4
_IU = np.triu_indices(NS, 1)

LEVEL_PARENTS = [NUM_LEAVES >> (l + 1) for l in range(12)]
LEVEL_OFF = [0]
for _n in LEVEL_PARENTS:
    LEVEL_OFF.append(LEVEL_OFF[-1] + _n)

NQ = 4
NG = 8
SPW = NUM_SITES // NG
LPQ = NUM_LEAVES // NQ
CH = 8
CL = LPQ // CH

CH_COL = [0, 64, 96]
QB_N = [64, 32, 16, 8, 4, 2, 1]
QB_RC = [(0, 0), (0, 64), (0, 96), (0, 112), (32, 0), (32, 16), (32, 32)]


def _cv(x):
    return jnp.full((16,), x, jnp.int32)


def _prep(rates, pi_logits, lengths, operations):
    pi = jax.nn.softmax(pi_logits)
    R = jnp.zeros((NS, NS), rates.dtype).at[_IU].set(rates)
    R = R + R.T
    Q = R * pi[None, :]
    Q = Q - jnp.diag(jnp.sum(Q, axis=1))
    mu = -jnp.sum(pi * jnp.diag(Q))
    Q = Q / mu
    dsq = jnp.sqrt(pi)
    S = (Q * dsq[:, None]) / dsq[None, :]
    S = 0.5 * (S + S.T)
    evals, evecs = jnp.linalg.eigh(S)
    left = evecs / dsq[:, None]
    right = evecs.T * dsq[None, :]
    P = jax.vmap(lambda t: left @ (jnp.exp(evals * t)[:, None] * right))(
        lax.stop_gradient(lengths))
    P = jnp.maximum(P, 0.0).astype(jnp.float32)

    flat = P.reshape(-1, 16)
    PL_parts, PR_parts = [], []
    for l in range(12):
        n = LEVEL_PARENTS[l]
        base = 0 if l == 0 else NUM_LEAVES + LEVEL_OFF[l - 1]
        seg = flat[base:base + 2 * n].reshape(n, 2, 16)
        PL_parts.append(seg[:, 0, :])
        PR_parts.append(seg[:, 1, :])
    PLf = jnp.concatenate(PL_parts, axis=0).T
    PRf = jnp.concatenate(PR_parts, axis=0).T

    cblocks = []
    for l in range(3):
        n = 64 >> l
        off = LEVEL_OFF[l]
        segL = PLf[:, off:off + (2048 >> l)].reshape(16, 4, CH, n)
        segR = PRf[:, off:off + (2048 >> l)].reshape(16, 4, CH, n)
        blk = jnp.concatenate([segL, segR], axis=0)
        cblocks.append(blk.transpose(1, 2, 0, 3))
    pqc = jnp.concatenate(cblocks, axis=3)
    pqc = jnp.pad(pqc, ((0, 0), (0, 0), (0, 0), (0, 16)))

    pqb = jnp.zeros((NQ, 64, 128), jnp.float32)
    for i, n in enumerate(QB_N):
        off = LEVEL_OFF[3 + i]
        segL = PLf[:, off:off + 4 * n].reshape(16, 4, n).transpose(1, 0, 2)
        segR = PRf[:, off:off + 4 * n].reshape(16, 4, n).transpose(1, 0, 2)
        blk = jnp.concatenate([segL, segR], axis=1)
        r0, c0 = QB_RC[i]
        pqb = pqb.at[:, r0:r0 + 32, c0:c0 + n].set(blk)

    def _stage_block(off, n):
        b = jnp.concatenate([PLf[:, off:off + n], PRf[:, off:off + n]],
                            axis=0)
        return jnp.pad(b, ((0, 0), (0, 16 - n)))

    pcs = jnp.zeros((32, 128), jnp.float32)
    pcs = pcs.at[:, 0:16].set(_stage_block(LEVEL_OFF[10], 2))
    pcs = pcs.at[:, 16:32].set(_stage_block(LEVEL_OFF[11], 1))
    pcs = pcs.at[0:NS, 32].set(pi)
    return pqc, pqb, pcs


def _do_level(pbuf, rbase, pcol0, npar, load_val, load_sc,
              store_val, store_sc):
    for gi in range(max(1, npar // 16)):
        p0 = gi * 16
        plv = [pbuf[rbase + r, pl.ds(pcol0 + p0, 16)] for r in range(16)]
        prv = [pbuf[rbase + 16 + r, pl.ds(pcol0 + p0, 16)]
               for r in range(16)]

        def body(t, p0=p0, plv=plv, prv=prv):
            ce = (p0 + lax.iota(jnp.int32, 16)) * 2
            lv = [load_val(t, j, ce) for j in range(NS)]
            rv = [load_val(t, j, ce + 1) for j in range(NS)]
            outs = []
            for s in range(NS):
                fl = plv[s * 4] * lv[0]
                fr = prv[s * 4] * rv[0]
                for j in range(1, NS):
                    fl = fl + plv[s * 4 + j] * lv[j]
                    fr = fr + prv[s * 4 + j] * rv[j]
                outs.append(fl * fr)
            m = jnp.maximum(jnp.maximum(outs[0], outs[1]),
                            jnp.maximum(outs[2], outs[3]))
            bits = plsc.bitcast(m, jnp.int32)
            eb = (bits >> 23) & 255
            ok = m > 0.0
            e = jnp.where(ok, eb - 127, 0)
            inv = jnp.where(ok, plsc.bitcast((254 - eb) << 23, jnp.float32),
                            jnp.float32(1.0))
            for s in range(NS):
                store_val(t, s, p0, outs[s] * inv)
            sc = e if load_sc is None else \
                load_sc(t, ce) + load_sc(t, ce + 1) + e
            store_sc(t, p0, sc)

        plsc.parallel_loop(0, SPW, unroll=1)(body)


def _vload(ref):
    return lambda t, j, cvec: plsc.load_gather(ref, [_cv(t + 32 * j), cvec])


def _sload(ref):
    return lambda t, cvec: plsc.load_gather(ref, [_cv(t), cvec])


def _vstore(ref, col0):
    def st(t, s, p0, x):
        ref[t + 32 * s, pl.ds(col0 + p0, 16)] = x
    return st


def _sstore(ref, col0):
    def st(t, p0, x):
        ref[t, pl.ds(col0 + p0, 16)] = x
    return st


def _root_vstore(ref):
    def st(t, s, p0, x):
        ref[t, pl.ds(16 * s, 16)] = x
    return st


def _root_sstore(ref):
    def st(t, p0, x):
        ref[t, pl.ds(64, 16)] = plsc.bitcast(x, jnp.float32)
    return st


def _body(leaf_hbm, pqc_hbm, pqb_hbm, pc_hbm, outv_hbm, outs_hbm,
          lbufa, lbufb, pbufc, pbufb, pcbuf, va, vb, acc, rootv,
          sa, sb, sacc, ov, osb, shv, sema, semb):
    c = lax.axis_index("c")
    s_id = lax.axis_index("s")
    q = s_id // 4
    g = c * 4 + (s_id % 4)
    site0 = g * SPW
    leaf0 = q * LPQ

    pltpu.sync_copy(pqb_hbm.at[q], pbufb)
    pltpu.sync_copy(pc_hbm, pcbuf)

    def leaf_slice(cc):
        return leaf_hbm.at[pl.ds(leaf0 + cc * CL, CL), g]

    def compute_chunk(cc, lbuf):
        def leaf_load(t, j, cvec):
            return plsc.load_gather(lbuf, [cvec, _cv(4 * t + j)])

        _do_level(pbufc, 0, CH_COL[0], 64,
                  leaf_load, None, _vstore(va, 0), _sstore(sa, 0))
        _do_level(pbufc, 0, CH_COL[1], 32,
                  _vload(va), _sload(sa), _vstore(vb, 0), _sstore(sb, 0))
        _do_level(pbufc, 0, CH_COL[2], 16,
                  _vload(vb), _sload(sb),
                  _vstore(acc, cc * 16), _sstore(sacc, cc * 16))

    pltpu.async_copy(leaf_slice(0), lbufa, sema)

    def chunk_pair(p, _):
        cc0 = 2 * p
        pltpu.async_copy(leaf_slice(cc0 + 1), lbufb, semb)
        pltpu.sync_copy(pqc_hbm.at[q, cc0], pbufc)
        pltpu.make_async_copy(leaf_slice(cc0), lbufa, sema).wait()
        compute_chunk(cc0, lbufa)

        @pl.when(p < CH // 2 - 1)
        def _prefetch():
            pltpu.async_copy(leaf_slice(cc0 + 2), lbufa, sema)

        pltpu.sync_copy(pqc_hbm.at[q, cc0 + 1], pbufc)
        pltpu.make_async_copy(leaf_slice(cc0 + 1), lbufb, semb).wait()
        compute_chunk(cc0 + 1, lbufb)
        return 0

    lax.fori_loop(0, CH // 2, chunk_pair, 0)

    chain = [(acc, sacc), (va, sa), (vb, sb), (va, sa),
             (vb, sb), (va, sa), (vb, sb)]
    for i, npar in enumerate(QB_N):
        src_v, src_s = chain[i]
        r0, c0 = QB_RC[i]
        if i < len(QB_N) - 1:
            dst_v, dst_s = chain[i + 1]
            _do_level(pbufb, r0, c0, npar, _vload(src_v), _sload(src_s),
                      _vstore(dst_v, 0), _sstore(dst_s, 0))
        else:
            _do_level(pbufb, r0, c0, npar, _vload(src_v), _sload(src_s),
                      _root_vstore(rootv), _root_sstore(rootv))

    pltpu.sync_copy(rootv, shv.at[s_id])
    plsc.subcore_barrier()

    @pl.when(s_id < 4)
    def _stage_c():
        for k in range(4):
            pltpu.sync_copy(shv.at[s_id + 4 * k], acc.at[pl.ds(32 * k, 32)])

        def spl(row, col):
            return plsc.load_gather(pcbuf, [_cv(row), _cv(col)])

        for h in range(2):
            tvec = lax.iota(jnp.int32, 16) + 16 * h
            vq = [[plsc.load_gather(acc, [tvec + 32 * k, _cv(16 * j)])
                   for j in range(NS)] for k in range(4)]
            scq = [plsc.bitcast(
                plsc.load_gather(acc, [tvec + 32 * k, _cv(64)]), jnp.int32)
                for k in range(4)]
            o1, s1 = [], []
            for i in range(2):
                row = []
                for s in range(NS):
                    fl = spl(s * 4, i) * vq[2 * i][0]
                    fr = spl(16 + s * 4, i) * vq[2 * i + 1][0]
                    for j in range(1, NS):
                        fl = fl + spl(s * 4 + j, i) * vq[2 * i][j]
                        fr = fr + spl(16 + s * 4 + j, i) * vq[2 * i + 1][j]
                    row.append(fl * fr)
                o1.append(row)
                s1.append(scq[2 * i] + scq[2 * i + 1])
            dot = jnp.zeros((16,), jnp.float32)
            for s in range(NS):
                fl = spl(s * 4, 16) * o1[0][0]
                fr = spl(16 + s * 4, 16) * o1[1][0]
                for j in range(1, NS):
                    fl = fl + spl(s * 4 + j, 16) * o1[0][j]
                    fr = fr + spl(16 + s * 4 + j, 16) * o1[1][j]
                dot = dot + spl(s, 32) * (fl * fr)
            ov[pl.ds(16 * h, 16)] = dot
            osb[pl.ds(16 * h, 16)] = (s1[0] + s1[1]).astype(jnp.float32)
        pltpu.sync_copy(ov, outv_hbm.at[pl.ds(site0, SPW)])
        pltpu.sync_copy(osb, outs_hbm.at[pl.ds(site0, SPW)])


def _scratch_types():
    f32, i32 = jnp.float32, jnp.int32
    return [
        pltpu.VMEM((CL, 128), f32),
        pltpu.VMEM((CL, 128), f32),
        pltpu.VMEM((32, 128), f32),
        pltpu.VMEM((64, 128), f32),
        pltpu.VMEM((32, 128), f32),
        pltpu.VMEM((128, 64), f32),
        pltpu.VMEM((128, 32), f32),
        pltpu.VMEM((128, 128), f32),
        pltpu.VMEM((32, 128), f32),
        pltpu.VMEM((32, 64), i32),
        pltpu.VMEM((32, 32), i32),
        pltpu.VMEM((32, 128), i32),
        pltpu.VMEM((SPW,), f32),
        pltpu.VMEM((SPW,), f32),
        pltpu.VMEM_SHARED((16, 32, 128), f32),
        pltpu.SemaphoreType.DMA,
        pltpu.SemaphoreType.DMA,
    ]


def kernel(leaf_data, rates, pi_logits, aligned_branch_lengths, operations):
    pqc, pqb, pcs = _prep(rates, pi_logits, aligned_branch_lengths,
                          operations)
    fn = pl.kernel(
        _body,
        out_type=[jax.ShapeDtypeStruct((NUM_SITES,), jnp.float32),
                  jax.ShapeDtypeStruct((NUM_SITES,), jnp.float32)],
        mesh=plsc.VectorSubcoreMesh(core_axis_name="c", subcore_axis_name="s"),
        scratch_types=_scratch_types(),
        compiler_params=pltpu.CompilerParams(needs_layout_passes=False),
    )
    leaf3 = leaf_data.reshape(NUM_LEAVES, NG, SPW * NS)
    val, sc = fn(leaf3, pqc, pqb, pcs)
    return jnp.mean(jnp.log(val) + sc * jnp.float32(np.log(2.0)))
